# baseline (speedup 1.0000x reference)
import functools

import jax
import jax.numpy as jnp
from jax import lax
from jax.experimental import pallas as pl
from jax.experimental.pallas import tpu as pltpu

B = 4
S = 8192
D = 1024
SUB = 8
LANE = 128
CH = 2048


def _tc_body(perm_ref, data_hbm, out_ref, stage, sem):
    @pl.when(pl.program_id(1) == 0)
    def _stage_batch():
        cp = pltpu.make_async_copy(data_hbm.at[pl.program_id(0)], stage, sem)
        cp.start()
        cp.wait()

    def step(i, carry):
        r = perm_ref[0, 0, i]
        out_ref[0, pl.ds(i, 1)] = stage[pl.ds(r, 1)]
        return carry

    lax.fori_loop(0, CH, step, 0, unroll=8)


_tc_gather = pl.pallas_call(
    _tc_body,
    grid=(B, S // CH),
    in_specs=[
        pl.BlockSpec((1, 1, CH), lambda b, j: (b * (S // CH) + j, 0, 0),
                     memory_space=pltpu.SMEM),
        pl.BlockSpec(memory_space=pl.ANY),
    ],
    out_specs=pl.BlockSpec((1, CH, SUB, LANE), lambda b, j: (b, j, 0, 0)),
    out_shape=jax.ShapeDtypeStruct((B, S, SUB, LANE), jnp.float32),
    scratch_shapes=[
        pltpu.VMEM((S, SUB, LANE), jnp.float32),
        pltpu.SemaphoreType.DMA,
    ],
    compiler_params=pltpu.CompilerParams(
        dimension_semantics=("arbitrary", "arbitrary"),
        vmem_limit_bytes=60 * 1024 * 1024,
    ),
)


def kernel(data, perm):
    out = _tc_gather(perm.reshape(B * (S // CH), 1, CH),
                     data.reshape(B, S, SUB, LANE))
    return out.reshape(B, S, D)


# speedup vs baseline: 1.1491x; 1.1491x over previous
import functools

import jax
import jax.numpy as jnp
from jax import lax
from jax.experimental import pallas as pl
from jax.experimental.pallas import tpu as pltpu
from jax.experimental.pallas import tpu_sc as plsc

B = 4
S = 8192
D = 1024
SUB = 8
LANE = 128
R = B * S

SC_BATCHES = 3
R_SC = SC_BATCHES * S
S_TC = R - R_SC

NC = 2
NS = 16
NW = NC * NS
RPW = R_SC // NW
C = 32
NCHUNK = RPW // C
NBUF = 3
L = 16

_mesh = plsc.VectorSubcoreMesh(core_axis_name="c", subcore_axis_name="s")


@functools.partial(
    pl.kernel,
    mesh=_mesh,
    out_type=jax.ShapeDtypeStruct((R_SC, D), jnp.float32),
    scratch_types=[
        pltpu.VMEM((RPW,), jnp.int32),
        pltpu.VMEM((NBUF * C, D), jnp.float32),
        pltpu.SemaphoreType.DMA,
        pltpu.SemaphoreType.DMA,
    ],
)
def _sc_gather(data_hbm, perm_hbm, out_hbm, idx_v, rows_v, gsem, ssem):
    wid = lax.axis_index("s") * NC + lax.axis_index("c")
    base = wid * RPW

    pltpu.sync_copy(perm_hbm.at[pl.ds(base, RPW)], idx_v)

    def _add_off(i, carry):
        sl = pl.ds(i * L, L)
        row_off = (base + i * L) & ~(S - 1)
        idx_v[sl] = idx_v[sl] + row_off
        return carry

    lax.fori_loop(0, RPW // L, _add_off, 0)

    def _buf(b):
        return rows_v.at[pl.ds(b * C, C)]

    def _gdesc(c, b):
        return pltpu.make_async_copy(data_hbm.at[idx_v.at[pl.ds(c * C, C)]],
                                     _buf(b), gsem)

    def _sdesc(c, b):
        return pltpu.make_async_copy(_buf(b),
                                     out_hbm.at[pl.ds(base + c * C, C)], ssem)

    def _step(c, carry):
        b = lax.rem(c, NBUF)

        @pl.when(c >= NBUF)
        def _wait_scatter():
            _sdesc(c - NBUF, b).wait()

        _gdesc(c, b).start()

        @pl.when(c >= 1)
        def _drain_prev():
            pb = lax.rem(c - 1, NBUF)
            _gdesc(c - 1, pb).wait()
            _sdesc(c - 1, pb).start()

        return carry

    lax.fori_loop(0, NCHUNK, _step, 0)

    last = NCHUNK - 1
    lb = last % NBUF
    _gdesc(last, lb).wait()
    _sdesc(last, lb).start()

    def _drain(i, carry):
        c = NCHUNK - NBUF + i
        _sdesc(c, lax.rem(c, NBUF)).wait()
        return carry

    lax.fori_loop(0, NBUF, _drain, 0)


CH = 2048


def _tc_body(perm_ref, data_hbm, out_ref, stage, sem):
    @pl.when(pl.program_id(0) == 0)
    def _stage_batch():
        cp = pltpu.make_async_copy(data_hbm.at[B - 1], stage, sem)
        cp.start()
        cp.wait()

    def step(i, carry):
        r = perm_ref[0, 0, i]
        out_ref[0, pl.ds(i, 1)] = stage[pl.ds(r, 1)]
        return carry

    lax.fori_loop(0, CH, step, 0, unroll=16)


_tc_gather = pl.pallas_call(
    _tc_body,
    grid=(S_TC // CH,),
    in_specs=[
        pl.BlockSpec((1, 1, CH),
                     lambda j: ((R - S_TC) // CH + j, 0, 0),
                     memory_space=pltpu.SMEM),
        pl.BlockSpec(memory_space=pl.ANY),
    ],
    out_specs=pl.BlockSpec((1, CH, SUB, LANE), lambda j: (0, j, 0, 0)),
    out_shape=jax.ShapeDtypeStruct((1, S_TC, SUB, LANE), jnp.float32),
    scratch_shapes=[
        pltpu.VMEM((S, SUB, LANE), jnp.float32),
        pltpu.SemaphoreType.DMA,
    ],
    compiler_params=pltpu.CompilerParams(
        dimension_semantics=("arbitrary",),
        vmem_limit_bytes=60 * 1024 * 1024,
    ),
)


def kernel(data, perm):
    sc_out = _sc_gather(data.reshape(R, D), perm.reshape(R))
    tc_out = _tc_gather(perm.reshape(R // CH, 1, CH),
                        data.reshape(B, S, SUB, LANE))
    return jnp.concatenate(
        [sc_out.reshape(SC_BATCHES, S, D), tc_out.reshape(1, S, D)],
        axis=0)
